# Initial kernel scaffold; baseline (speedup 1.0000x reference)
import jax
import jax.numpy as jnp
from jax.experimental import pallas as pl


def kernel(x_hat, lam_hat, A_rows, A_cols, A_vals, b_pad, c_pad, b_mask, c_mask):
    raise NotImplementedError("write your pallas kernel here")


# speedup vs baseline: 208.7333x; 208.7333x over previous
import functools

import jax
import jax.numpy as jnp
from jax import lax
from jax.experimental import pallas as pl
from jax.experimental.pallas import tpu as pltpu
from jax.experimental.pallas import tpu_sc as plsc

B_ = 4
M_ = 16384
N_ = 16384
NNZ_ = 262144
W_PRIMAL, W_DUAL, W_STAT, W_COMP = 0.1, 0.1, 0.6, 0.2

NC = 2
NS = 16
L = 16

SLICE = M_ // NS
ROWS_ALL = NNZ_ // 128
ROWS_PER_TILE = ROWS_ALL // NS
KC = 16
CHUNK = KC * 128
NCHUNK = ROWS_PER_TILE // KC


def _body(x_ref, lam_ref, rows_ref, cols_ref, vals_ref, b_ref, c_ref,
          out_ref,
          x_v, lam_v, rows_v, cols_v, vals_v, prod_v, prod2_v,
          ax_v, atl_v, b_v, c_v, lamc_v, loss_v, zero_v,
          ax_s, atl_s):
    c = lax.axis_index("c")
    s = lax.axis_index("s")
    zf = jnp.zeros((L,), jnp.float32)

    def zinit(k, carry):
        zero_v[pl.ds(k * L, L)] = zf
        return carry

    lax.fori_loop(0, SLICE // L, zinit, 0)
    for l in range(4):
        loss_v[l, :] = zf

    for bi in range(2):
        i = c * 2 + bi
        pltpu.sync_copy(x_ref.at[i], x_v)
        pltpu.sync_copy(lam_ref.at[i], lam_v)
        pltpu.sync_copy(zero_v, ax_s.at[pl.ds(s * SLICE, SLICE)])
        pltpu.sync_copy(zero_v, atl_s.at[pl.ds(s * SLICE, SLICE)])
        plsc.subcore_barrier()

        for ch in range(NCHUNK):
            e0 = (s * ROWS_PER_TILE + ch * KC) * 128
            pltpu.sync_copy(rows_ref.at[i, pl.ds(e0, CHUNK)], rows_v)
            pltpu.sync_copy(cols_ref.at[i, pl.ds(e0, CHUNK)], cols_v)
            pltpu.sync_copy(vals_ref.at[i, pl.ds(e0, CHUNK)], vals_v)

            def chunk_body(k, carry):
                cvec = cols_v[pl.ds(k * L, L)]
                rvec = rows_v[pl.ds(k * L, L)]
                vvec = vals_v[pl.ds(k * L, L)]
                xg = plsc.load_gather(x_v, [cvec])
                lg = plsc.load_gather(lam_v, [rvec])
                prod_v[pl.ds(k * L, L)] = vvec * xg
                prod2_v[pl.ds(k * L, L)] = vvec * lg
                return carry

            lax.fori_loop(0, CHUNK // L, chunk_body, 0)
            pltpu.sync_copy(prod_v, ax_s.at[rows_v], add=True)
            pltpu.sync_copy(prod2_v, atl_s.at[cols_v], add=True)

        plsc.subcore_barrier()

        off = s * SLICE
        pltpu.sync_copy(ax_s.at[pl.ds(off, SLICE)], ax_v)
        pltpu.sync_copy(atl_s.at[pl.ds(off, SLICE)], atl_v)
        pltpu.sync_copy(b_ref.at[i, pl.ds(off, SLICE)], b_v)
        pltpu.sync_copy(c_ref.at[i, pl.ds(off, SLICE)], c_v)
        pltpu.sync_copy(lam_ref.at[i, pl.ds(off, SLICE)], lamc_v)

        def loss_body(k, accs):
            ap, ad, ast, ac = accs
            ax = ax_v[pl.ds(k * L, L)]
            bb = b_v[pl.ds(k * L, L)]
            ll = lamc_v[pl.ds(k * L, L)]
            at = atl_v[pl.ds(k * L, L)]
            cc = c_v[pl.ds(k * L, L)]
            r = ax - bb
            p = jnp.maximum(r, 0.0)
            dn = jnp.maximum(-ll, 0.0)
            st = at + cc
            cm = ll * r
            return (ap + p * p, ad + dn * dn, ast + st * st, ac + cm * cm)

        ap, ad, ast, ac = lax.fori_loop(0, SLICE // L, loss_body,
                                        (zf, zf, zf, zf))
        loss_v[0, :] = loss_v[0, :] + ap
        loss_v[1, :] = loss_v[1, :] + ad
        loss_v[2, :] = loss_v[2, :] + ast
        loss_v[3, :] = loss_v[3, :] + ac
        plsc.subcore_barrier()

    pltpu.sync_copy(loss_v, out_ref.at[c, s])


_sc_kernel = functools.partial(
    pl.kernel,
    out_type=jax.ShapeDtypeStruct((NC, NS, 4, L), jnp.float32),
    mesh=plsc.VectorSubcoreMesh(core_axis_name="c", subcore_axis_name="s"),
    compiler_params=pltpu.CompilerParams(needs_layout_passes=False),
    scratch_types=[
        pltpu.VMEM((N_,), jnp.float32),
        pltpu.VMEM((M_,), jnp.float32),
        pltpu.VMEM((CHUNK,), jnp.int32),
        pltpu.VMEM((CHUNK,), jnp.int32),
        pltpu.VMEM((CHUNK,), jnp.float32),
        pltpu.VMEM((CHUNK,), jnp.float32),
        pltpu.VMEM((CHUNK,), jnp.float32),
        pltpu.VMEM((SLICE,), jnp.float32),
        pltpu.VMEM((SLICE,), jnp.float32),
        pltpu.VMEM((SLICE,), jnp.float32),
        pltpu.VMEM((SLICE,), jnp.float32),
        pltpu.VMEM((SLICE,), jnp.float32),
        pltpu.VMEM((4, L), jnp.float32),
        pltpu.VMEM((SLICE,), jnp.float32),
        pltpu.VMEM_SHARED((M_,), jnp.float32),
        pltpu.VMEM_SHARED((N_,), jnp.float32),
    ],
)(_body)


def kernel(x_hat, lam_hat, A_rows, A_cols, A_vals, b_pad, c_pad, b_mask, c_mask):
    x2 = x_hat.astype(jnp.float32).reshape(B_, N_)
    lam2 = lam_hat.astype(jnp.float32).reshape(B_, M_)
    rows3 = A_rows.astype(jnp.int32)
    cols3 = A_cols.astype(jnp.int32)
    vals3 = A_vals.astype(jnp.float32)
    part = _sc_kernel(x2, lam2, rows3, cols3, vals3,
                      b_pad.astype(jnp.float32), c_pad.astype(jnp.float32))
    sums = part.sum(axis=(0, 1, 3))
    total = (W_PRIMAL * sums[0] / M_ + W_DUAL * sums[1] / M_
             + W_STAT * sums[2] / N_ + W_COMP * sums[3] / M_) / B_
    return total.astype(jnp.float32)
